# baseline (speedup 1.0000x reference)
import functools
import jax
import jax.numpy as jnp
from jax import lax
from jax.experimental import pallas as pl
from jax.experimental.pallas import tpu as pltpu
from jax.experimental.pallas import tpu_sc as plsc

_B = 4
_T = 8192
_C = 1024
_K = 8

_NC = 2
_NS = 16
_NW = _NC * _NS
_CPW = _C // (_NW // _B)
_LG = _CPW // 16
_TC = 256
_NCHUNK = _T // _TC


_SORT8 = [(0, 1), (2, 3), (4, 5), (6, 7),
          (0, 2), (1, 3), (4, 6), (5, 7),
          (1, 2), (5, 6),
          (0, 4), (1, 5), (2, 6), (3, 7),
          (2, 4), (3, 5),
          (1, 2), (3, 4), (5, 6)]
_MERGE8 = [(0, 4), (1, 5), (2, 6), (3, 7),
           (0, 2), (1, 3), (4, 6), (5, 7),
           (0, 1), (2, 3), (4, 5), (6, 7)]


def _merge_sorted8(s, r):
    s = [jnp.maximum(s[i], r[_K - 1 - i]) for i in range(_K)]
    for (i, j) in _MERGE8:
        s[i], s[j] = jnp.maximum(s[i], s[j]), jnp.minimum(s[i], s[j])
    return s


def _insert_chunk(buf, cand, state):
    iota = lax.iota(jnp.int32, 16)
    neg_inf = jnp.full((16,), -jnp.inf, dtype=jnp.float32)
    sixteen = jnp.full((16,), 16, dtype=jnp.int32)
    zero = jnp.zeros((16,), dtype=jnp.int32)
    neg_one = jnp.full((16,), -1, dtype=jnp.int32)

    new_state = []
    for l in range(_LG):
        s = state[l]
        thr = s[_K - 1]
        cols = jnp.full((16,), 16 * l, dtype=jnp.int32) + iota

        def rowf(g, idx):
            t = g * _K
            vs = [buf[t + i, pl.ds(16 * l, 16)] for i in range(_K)]
            lvl = vs
            while len(lvl) > 1:
                lvl = [jnp.maximum(lvl[i], lvl[i + 1])
                       for i in range(0, len(lvl), 2)]
            m = lvl[0] > thr
            tvec = jnp.full((16,), 0, dtype=jnp.int32) + t
            plsc.store_scatter(cand, [idx], tvec, mask=m)
            return idx + jnp.where(m, sixteen, zero)

        idx = lax.fori_loop(0, _TC // _K, rowf, iota, unroll=2)

        cnt = (idx - iota) >> 4
        scnt = plsc.sort_key_val(cnt, cnt, descending=True)[0]
        mc = scnt[0]

        def entryf(i, s):
            t0 = cand[pl.ds(i * 16, 16)]
            cand[pl.ds(i * 16, 16)] = neg_one
            valid = t0 >= 0
            t0c = jnp.maximum(t0, 0)
            r = []
            for j in range(_K):
                v = plsc.load_gather(buf, [t0c + j, cols], mask=valid)
                r.append(jnp.where(valid, v, neg_inf))
            for (a, b) in _SORT8:
                r[a], r[b] = jnp.maximum(r[a], r[b]), jnp.minimum(r[a], r[b])
            return tuple(_merge_sorted8(list(s), r))

        s = lax.fori_loop(0, mc, entryf, s)
        new_state.append(s)
    return tuple(new_state)


def _sc_body(in_hbm, out_hbm, buf0, buf1, cand, obuf, sem0, sem1):
    wid = lax.axis_index("s") * _NC + lax.axis_index("c")
    b = wid // (_NW // _B)
    c0 = (wid % (_NW // _B)) * _CPW

    bufs = (buf0, buf1)
    sems = (sem0, sem1)

    def dma(chunk, slot):
        t0 = jnp.minimum(chunk, _NCHUNK - 1) * _TC
        return pltpu.make_async_copy(
            in_hbm.at[b, pl.ds(t0, _TC), pl.ds(c0, _CPW)],
            bufs[slot], sems[slot])

    neg_inf = jnp.full((16,), -jnp.inf, dtype=jnp.float32)
    init = tuple(tuple(neg_inf for _ in range(_K)) for _ in range(_LG))

    dma(0, 0).start()
    dma(1, 1).start()

    neg_one = jnp.full((16,), -1, dtype=jnp.int32)

    @pl.loop(0, _TC // _K)
    def _(i):
        cand[pl.ds(i * 16, 16)] = neg_one

    @pl.loop(0, _NCHUNK, step=2, init_carry=init)
    def state(chunk, state):
        for slot in range(2):
            dma(chunk + slot, slot).wait()
            state = _insert_chunk(bufs[slot], cand, state)
            dma(chunk + slot + 2, slot).start()
        return state

    dma(_NCHUNK, 0).wait()
    dma(_NCHUNK + 1, 1).wait()

    for l in range(_LG):
        for j in range(_K):
            obuf[j, pl.ds(16 * l, 16)] = state[l][j]

    pltpu.sync_copy(obuf, out_hbm.at[b, :, pl.ds(c0, _CPW)])


@jax.jit
def _kmax_sc(inputs):
    mesh = plsc.VectorSubcoreMesh(
        core_axis_name="c", subcore_axis_name="s",
        num_cores=_NC, num_subcores=_NS)
    kern = pl.kernel(
        _sc_body,
        out_type=jax.ShapeDtypeStruct((_B, _K, _C), jnp.float32),
        mesh=mesh,
        compiler_params=pltpu.CompilerParams(needs_layout_passes=False),
        scratch_types=[
            pltpu.VMEM((_TC, _CPW), jnp.float32),
            pltpu.VMEM((_TC, _CPW), jnp.float32),
            pltpu.VMEM((_TC // _K * 16,), jnp.int32),
            pltpu.VMEM((_K, _CPW), jnp.float32),
            pltpu.SemaphoreType.DMA,
            pltpu.SemaphoreType.DMA,
        ],
    )
    return kern(inputs)


def kernel(inputs):
    out = _kmax_sc(inputs)
    return out.transpose(0, 2, 1).reshape(_B, _C * _K)


# speedup vs baseline: 1.2847x; 1.2847x over previous
import functools
import jax
import jax.numpy as jnp
from jax import lax
from jax.experimental import pallas as pl
from jax.experimental.pallas import tpu as pltpu
from jax.experimental.pallas import tpu_sc as plsc

_B = 4
_T = 8192
_C = 1024
_K = 8

_NC = 2
_NS = 16
_NW = _NC * _NS
_CPW = _C // (_NW // _B)
_LG = _CPW // 16
_TC = 256
_NCHUNK = _T // _TC


_SORT8 = [(0, 1), (2, 3), (4, 5), (6, 7),
          (0, 2), (1, 3), (4, 6), (5, 7),
          (1, 2), (5, 6),
          (0, 4), (1, 5), (2, 6), (3, 7),
          (2, 4), (3, 5),
          (1, 2), (3, 4), (5, 6)]
_MERGE8 = [(0, 4), (1, 5), (2, 6), (3, 7),
           (0, 2), (1, 3), (4, 6), (5, 7),
           (0, 1), (2, 3), (4, 5), (6, 7)]


def _insert_chunk(buf, state):
    new_state = []
    for l in range(_LG):
        s = state[l]

        def batch_body(g, s):
            t = g * _K
            r = [buf[t + i, pl.ds(16 * l, 16)] for i in range(_K)]
            for (i, j) in _SORT8:
                r[i], r[j] = jnp.maximum(r[i], r[j]), jnp.minimum(r[i], r[j])
            s = [jnp.maximum(s[i], r[_K - 1 - i]) for i in range(_K)]
            for (i, j) in _MERGE8:
                s[i], s[j] = jnp.maximum(s[i], s[j]), jnp.minimum(s[i], s[j])
            return tuple(s)

        s = lax.fori_loop(0, _TC // _K, batch_body, s)
        new_state.append(s)
    return tuple(new_state)


def _sc_body(in_hbm, out_hbm, buf0, buf1, obuf, sem0, sem1):
    wid = lax.axis_index("s") * _NC + lax.axis_index("c")
    b = wid // (_NW // _B)
    c0 = (wid % (_NW // _B)) * _CPW

    bufs = (buf0, buf1)
    sems = (sem0, sem1)

    def dma(chunk, slot):
        t0 = jnp.minimum(chunk, _NCHUNK - 1) * _TC
        return pltpu.make_async_copy(
            in_hbm.at[b, pl.ds(t0, _TC), pl.ds(c0, _CPW)],
            bufs[slot], sems[slot])

    neg_inf = jnp.full((16,), -jnp.inf, dtype=jnp.float32)
    init = tuple(tuple(neg_inf for _ in range(_K)) for _ in range(_LG))

    dma(0, 0).start()
    dma(1, 1).start()

    @pl.loop(0, _NCHUNK, step=2, init_carry=init)
    def state(chunk, state):
        for slot in range(2):
            dma(chunk + slot, slot).wait()
            state = _insert_chunk(bufs[slot], state)
            dma(chunk + slot + 2, slot).start()
        return state

    dma(_NCHUNK, 0).wait()
    dma(_NCHUNK + 1, 1).wait()

    for l in range(_LG):
        for j in range(_K):
            obuf[j, pl.ds(16 * l, 16)] = state[l][j]

    pltpu.sync_copy(obuf, out_hbm.at[b, :, pl.ds(c0, _CPW)])


@jax.jit
def _kmax_sc(inputs):
    mesh = plsc.VectorSubcoreMesh(
        core_axis_name="c", subcore_axis_name="s",
        num_cores=_NC, num_subcores=_NS)
    kern = pl.kernel(
        _sc_body,
        out_type=jax.ShapeDtypeStruct((_B, _K, _C), jnp.float32),
        mesh=mesh,
        compiler_params=pltpu.CompilerParams(needs_layout_passes=False),
        scratch_types=[
            pltpu.VMEM((_TC, _CPW), jnp.float32),
            pltpu.VMEM((_TC, _CPW), jnp.float32),
            pltpu.VMEM((_K, _CPW), jnp.float32),
            pltpu.SemaphoreType.DMA,
            pltpu.SemaphoreType.DMA,
        ],
    )
    return kern(inputs)


def kernel(inputs):
    out = _kmax_sc(inputs)
    return out.transpose(0, 2, 1).reshape(_B, _C * _K)
